# baseline (speedup 1.0000x reference)
import functools

import jax
import jax.numpy as jnp
from jax import lax
from jax.experimental import pallas as pl
from jax.experimental.pallas import tpu as pltpu
from jax.experimental.pallas import tpu_sc as plsc

U = 128
E = 160000
N = 10000
NPAD = 10240
NB = 128
NNB = NPAD // NB
B = 640
NEB = E // B
W = NEB + NNB - 1
NB2 = 200
NPB = 1280
CUTOFF = 5.0

SC_NC = 2
SC_NS = 16
SC_NW = SC_NC * SC_NS
EPW = E // SC_NW
SC_CH = 128
SC_NFULL = EPW // SC_CH
SC_TAIL = EPW - SC_NFULL * SC_CH


def _node_tables(z8, emb_pad, Wemb2):
    def body(z_ref, emb_ref, w_ref, xl_ref, xr_ref):
        ew_l = lax.dot_general(emb_ref[...], w_ref[:, :U], (((1,), (1,)), ((), ())),
                               preferred_element_type=jnp.float32)
        ew_r = lax.dot_general(emb_ref[...], w_ref[:, U:], (((1,), (1,)), ((), ())),
                               preferred_element_type=jnp.float32)
        zrow = z_ref[0:1, :]
        tids = lax.broadcasted_iota(jnp.int32, (U, NPB), 0).astype(jnp.float32)
        P = (tids == zrow).astype(jnp.float32)
        xl_ref[...] = lax.dot_general(P, ew_l, (((0,), (0,)), ((), ())),
                                      preferred_element_type=jnp.float32)
        xr_ref[...] = lax.dot_general(P, ew_r, (((0,), (0,)), ((), ())),
                                      preferred_element_type=jnp.float32)
    return pl.pallas_call(
        body,
        grid=(NPAD // NPB,),
        in_specs=[
            pl.BlockSpec((8, NPB), lambda i: (0, i)),
            pl.BlockSpec((U, U), lambda i: (0, 0)),
            pl.BlockSpec((U, 2 * U), lambda i: (0, 0)),
        ],
        out_specs=[pl.BlockSpec((NPB, U), lambda i: (i, 0)),
                   pl.BlockSpec((NPB, U), lambda i: (i, 0))],
        out_shape=[jax.ShapeDtypeStruct((NPAD, U), jnp.float32),
                   jax.ShapeDtypeStruct((NPAD, U), jnp.float32)],
    )(z8, emb_pad, Wemb2)


def _sc_gather(src, dst, xl, xr):
    mesh = plsc.VectorSubcoreMesh(core_axis_name="c", subcore_axis_name="s")

    @functools.partial(
        pl.kernel,
        out_type=jax.ShapeDtypeStruct((E, U), jnp.float32),
        mesh=mesh,
        scratch_types=[
            pltpu.VMEM((EPW,), jnp.int32),
            pltpu.VMEM((EPW,), jnp.int32),
            pltpu.VMEM((SC_CH, U), jnp.float32),
            pltpu.VMEM((SC_CH, U), jnp.float32),
            pltpu.VMEM((SC_CH, U), jnp.float32),
            pltpu.VMEM((SC_CH, U), jnp.float32),
            pltpu.VMEM((SC_TAIL, U), jnp.float32),
            pltpu.VMEM((SC_TAIL, U), jnp.float32),
            pltpu.SemaphoreType.DMA,
            pltpu.SemaphoreType.DMA,
            pltpu.SemaphoreType.DMA,
            pltpu.SemaphoreType.DMA,
        ],
    )
    def k(src_hbm, dst_hbm, xl_hbm, xr_hbm, out_hbm,
          is_v, id_v, al, ar, bl, br, tl, tr, sAl, sAr, sBl, sBr):
        wid = lax.axis_index("s") * SC_NC + lax.axis_index("c")
        base = wid * EPW
        pltpu.sync_copy(src_hbm.at[pl.ds(base, EPW)], is_v)
        pltpu.sync_copy(dst_hbm.at[pl.ds(base, EPW)], id_v)

        def fire(c, bufl, bufr, sl, sr):
            off = c * SC_CH
            pltpu.async_copy(xl_hbm.at[is_v.at[pl.ds(off, SC_CH)]], bufl, sl)
            pltpu.async_copy(xr_hbm.at[id_v.at[pl.ds(off, SC_CH)]], bufr, sr)

        def addrows(bufl, bufr, n):
            def row(i, _):
                for c in range(U // 16):
                    s = pl.ds(c * 16, 16)
                    bufl[i, s] = bufl[i, s] + bufr[i, s]
                return 0
            lax.fori_loop(0, n, row, 0)

        def drain_store(c, bufl, bufr, sl, sr):
            off = c * SC_CH
            pltpu.make_async_copy(xl_hbm.at[is_v.at[pl.ds(off, SC_CH)]], bufl, sl).wait()
            pltpu.make_async_copy(xr_hbm.at[id_v.at[pl.ds(off, SC_CH)]], bufr, sr).wait()
            addrows(bufl, bufr, SC_CH)
            pltpu.sync_copy(bufl, out_hbm.at[pl.ds(base + off, SC_CH)])

        fire(0, al, ar, sAl, sAr)

        def body(tt, _):
            e = 2 * tt
            fire(e + 1, bl, br, sBl, sBr)
            drain_store(e, al, ar, sAl, sAr)
            fire(e + 2, al, ar, sAl, sAr)
            drain_store(e + 1, bl, br, sBl, sBr)
            return 0

        lax.fori_loop(0, (SC_NFULL - 1) // 2, body, 0)
        drain_store(SC_NFULL - 1, al, ar, sAl, sAr)
        off_t = SC_NFULL * SC_CH
        pltpu.async_copy(xl_hbm.at[is_v.at[pl.ds(off_t, SC_TAIL)]], tl, sAl)
        pltpu.async_copy(xr_hbm.at[id_v.at[pl.ds(off_t, SC_TAIL)]], tr, sAr)
        pltpu.make_async_copy(xl_hbm.at[is_v.at[pl.ds(off_t, SC_TAIL)]], tl, sAl).wait()
        pltpu.make_async_copy(xr_hbm.at[id_v.at[pl.ds(off_t, SC_TAIL)]], tr, sAr).wait()
        addrows(tl, tr, SC_TAIL)
        pltpu.sync_copy(tl, out_hbm.at[pl.ds(base + off_t, SC_TAIL)])

    return k(src, dst, xl, xr)


def _agg_body(nbm_ref, ebm_ref, eattr_ref, sp_ref, zp_ref,
              wdist_ref, bdist_ref, bemb2_ref, out_ref):
    w = pl.program_id(0)
    nb = nbm_ref[w]
    prev = nbm_ref[jnp.maximum(w - 1, 0)]

    @pl.when(jnp.logical_or(w == 0, nb != prev))
    def _():
        out_ref[...] = jnp.zeros_like(out_ref)

    sp = sp_ref[...]
    vx0, vy0, vz0 = sp[0:1, :], sp[1:2, :], sp[2:3, :]
    wgt, dstf = sp[3:4, :], sp[4:5, :]
    inv = 1.0 / jnp.maximum(jnp.sqrt(vx0 * vx0 + vy0 * vy0 + vz0 * vz0), 1e-6)
    vx, vy, vz = vx0 * inv, vy0 * inv, vz0 * inv
    C = 0.5 * (jnp.cos(jnp.pi / CUTOFF * wgt) + 1.0) * (wgt < CUTOFF)
    sxx, syy, szz = vx * vx, vy * vy, vz * vz
    tr3 = (sxx + syy + szz) * (1.0 / 3.0)
    cs = (C, C * vx, C * vy, C * vz,
          C * (sxx - tr3), C * (vx * vy), C * (vx * vz),
          C * (syy - tr3), C * (vy * vz), C * (szz - tr3))

    ea = lax.dot_general(eattr_ref[...], wdist_ref[...], (((1,), (1,)), ((), ())),
                         preferred_element_type=jnp.float32) + bdist_ref[0:1, :]
    Zij = zp_ref[...] + bemb2_ref[0:1, :]
    bf16 = jnp.bfloat16
    f0 = (ea[:, :U] * Zij).astype(bf16)
    f1 = (ea[:, U:2 * U] * Zij).astype(bf16)
    f2 = (ea[:, 2 * U:] * Zij).astype(bf16)

    ids = (nb * NB + lax.broadcasted_iota(jnp.int32, (NB, B), 0)).astype(jnp.float32)
    hit = ids == dstf
    P0 = jnp.where(hit, cs[0], 0.0).astype(bf16)
    P1 = jnp.concatenate([jnp.where(hit, cs[k], 0.0) for k in (1, 2, 3)],
                         axis=0).astype(bf16)
    P2 = jnp.concatenate([jnp.where(hit, cs[k], 0.0) for k in range(4, 10)],
                         axis=0).astype(bf16)
    dn = (((1,), (0,)), ((), ()))
    out_ref[0, :NB, :] += lax.dot_general(P0, f0, dn,
                                          preferred_element_type=jnp.float32)
    out_ref[0, NB:4 * NB, :] += lax.dot_general(P1, f1, dn,
                                                preferred_element_type=jnp.float32)
    out_ref[0, 4 * NB:, :] += lax.dot_general(P2, f2, dn,
                                              preferred_element_type=jnp.float32)


def _aggregate(nb_map, eb_map, eattr, sp, zp, Wdist, bdist2, bemb22):
    grid_spec = pltpu.PrefetchScalarGridSpec(
        num_scalar_prefetch=2,
        grid=(W,),
        in_specs=[
            pl.BlockSpec((B, 16), lambda w, nbm, ebm: (ebm[w], 0)),
            pl.BlockSpec((8, B), lambda w, nbm, ebm: (0, ebm[w])),
            pl.BlockSpec((B, U), lambda w, nbm, ebm: (ebm[w], 0)),
            pl.BlockSpec((3 * U, 16), lambda w, nbm, ebm: (0, 0)),
            pl.BlockSpec((8, 3 * U), lambda w, nbm, ebm: (0, 0)),
            pl.BlockSpec((8, U), lambda w, nbm, ebm: (0, 0)),
        ],
        out_specs=pl.BlockSpec((1, 10 * NB, U), lambda w, nbm, ebm: (nbm[w], 0, 0)),
    )
    return pl.pallas_call(
        _agg_body,
        grid_spec=grid_spec,
        out_shape=jax.ShapeDtypeStruct((NNB, 10 * NB, U), jnp.float32),
        compiler_params=pltpu.CompilerParams(dimension_semantics=("arbitrary",)),
    )(nb_map, eb_map, eattr, sp, zp, Wdist, bdist2, bemb22)


def _final_body(agg_ref, wt0_ref, wt1_ref, wt2_ref, ws1_ref, bs1_ref,
                ws2_ref, bs2_ref, lng_ref, lnb_ref, out_ref):
    a = agg_ref[0]
    Isum = a[:NB]
    ax, ay, az = a[NB:2 * NB], a[2 * NB:3 * NB], a[3 * NB:4 * NB]
    sxx, sxy, sxz = a[4 * NB:5 * NB], a[5 * NB:6 * NB], a[6 * NB:7 * NB]
    syy, syz, szz = a[7 * NB:8 * NB], a[8 * NB:9 * NB], a[9 * NB:]

    x00, x11, x22 = Isum + sxx, Isum + syy, Isum + szz
    nrm = (x00 * x00 + x11 * x11 + x22 * x22
           + (sxy - az) ** 2 + (sxy + az) ** 2
           + (sxz + ay) ** 2 + (sxz - ay) ** 2
           + (syz - ax) ** 2 + (syz + ax) ** 2)
    mu = jnp.mean(nrm, axis=1, keepdims=True)
    var = jnp.mean((nrm - mu) ** 2, axis=1, keepdims=True)
    h = (nrm - mu) / jnp.sqrt(var + 1e-5) * lng_ref[0:1, :] + lnb_ref[0:1, :]

    h1 = lax.dot_general(h, ws1_ref[...], (((1,), (1,)), ((), ())),
                         preferred_element_type=jnp.float32) + bs1_ref[0:1, :]
    h1 = h1 * (1.0 / (1.0 + jnp.exp(-h1)))
    h2 = lax.dot_general(h1, ws2_ref[...], (((1,), (1,)), ((), ())),
                         preferred_element_type=jnp.float32) + bs2_ref[0:1, :]
    h2 = h2 * (1.0 / (1.0 + jnp.exp(-h2)))
    nI, nA, nS = h2[:, :U], h2[:, U:2 * U], h2[:, 2 * U:]

    tI = lax.dot_general(Isum, wt0_ref[...], (((1,), (1,)), ((), ())),
                         preferred_element_type=jnp.float32)
    tA = lax.dot_general(a[NB:4 * NB], wt1_ref[...], (((1,), (1,)), ((), ())),
                         preferred_element_type=jnp.float32)
    tax, tay, taz = tA[:NB], tA[NB:2 * NB], tA[2 * NB:]
    tS = lax.dot_general(a[4 * NB:], wt2_ref[...], (((1,), (1,)), ((), ())),
                         preferred_element_type=jnp.float32)
    tsxx, tsxy, tsxz = tS[:NB], tS[NB:2 * NB], tS[2 * NB:3 * NB]
    tsyy, tsyz, tszz = tS[3 * NB:4 * NB], tS[4 * NB:5 * NB], tS[5 * NB:]

    dI = tI * nI
    out_ref[...] = jnp.concatenate([
        dI + tsxx * nS,
        -taz * nA + tsxy * nS,
        tay * nA + tsxz * nS,
        taz * nA + tsxy * nS,
        dI + tsyy * nS,
        -tax * nA + tsyz * nS,
        -tay * nA + tsxz * nS,
        tax * nA + tsyz * nS,
        dI + tszz * nS,
    ], axis=1)


def _finalize(agg, Wt0, Wt1, Wt2, Ws1, bs12, Ws2p, bs2p2, lng2, lnb2):
    return pl.pallas_call(
        _final_body,
        grid=(NNB,),
        in_specs=[
            pl.BlockSpec((1, 10 * NB, U), lambda i: (i, 0, 0)),
            pl.BlockSpec((U, U), lambda i: (0, 0)),
            pl.BlockSpec((U, U), lambda i: (0, 0)),
            pl.BlockSpec((U, U), lambda i: (0, 0)),
            pl.BlockSpec((2 * U, U), lambda i: (0, 0)),
            pl.BlockSpec((8, 2 * U), lambda i: (0, 0)),
            pl.BlockSpec((3 * U, 2 * U), lambda i: (0, 0)),
            pl.BlockSpec((8, 3 * U), lambda i: (0, 0)),
            pl.BlockSpec((8, U), lambda i: (0, 0)),
            pl.BlockSpec((8, U), lambda i: (0, 0)),
        ],
        out_specs=pl.BlockSpec((NB, 9 * U), lambda i: (i, 0)),
        out_shape=jax.ShapeDtypeStruct((NPAD, 9 * U), jnp.float32),
    )(agg, Wt0, Wt1, Wt2, Ws1, bs12, Ws2p, bs2p2, lng2, lnb2)


def kernel(z, edge_index, edge_weight, edge_vec, edge_attr, col_data,
           col_indptr, emb, Wdist, bdist, Wemb2, bemb2, Wt0, Wt1, Wt2,
           Ws1, bs1, Ws2, bs2, ln_g, ln_b):
    f32 = jnp.float32
    src = edge_index[0].astype(jnp.int32)
    dst = edge_index[1].astype(jnp.int32)

    emb_pad = jnp.pad(emb.astype(f32), ((0, U - emb.shape[0]), (0, 0)))
    zp = jnp.pad(z.astype(f32), (0, NPAD - N))
    z8 = jnp.broadcast_to(zp[None, :], (8, NPAD))
    xl, xr = _node_tables(z8, emb_pad, Wemb2.astype(f32))
    zp = _sc_gather(src, dst, xl, xr)

    sp = jnp.concatenate([
        edge_vec.astype(f32).T,
        edge_weight.astype(f32)[None, :],
        dst.astype(f32)[None, :],
        jnp.zeros((3, E), f32),
    ], axis=0)

    lnb = dst[B - 1::B] // NB
    b = lnb.at[-1].set(NNB - 1)
    bprev = jnp.concatenate([jnp.zeros((1,), jnp.int32), b[:-1]])
    cnt = b - bprev + 1
    eb_map = jnp.repeat(jnp.arange(NEB, dtype=jnp.int32), cnt,
                        total_repeat_length=W)
    start = jnp.cumsum(cnt) - cnt
    nb_map = (bprev[eb_map] + jnp.arange(W, dtype=jnp.int32)
              - start[eb_map]).astype(jnp.int32)

    bdist2 = jnp.broadcast_to(bdist.astype(f32), (8, 3 * U))
    bemb22 = jnp.broadcast_to(bemb2.astype(f32), (8, U))
    agg = _aggregate(nb_map, eb_map, edge_attr.astype(f32), sp, zp,
                     Wdist.astype(f32), bdist2, bemb22)

    perm = jnp.concatenate([jnp.arange(U) * 3, jnp.arange(U) * 3 + 1,
                            jnp.arange(U) * 3 + 2])
    Ws2p = Ws2[perm].astype(f32)
    bs2p2 = jnp.broadcast_to(bs2[perm].astype(f32), (8, 3 * U))
    bs12 = jnp.broadcast_to(bs1.astype(f32), (8, 2 * U))
    lng2 = jnp.broadcast_to(ln_g.astype(f32), (8, U))
    lnb2 = jnp.broadcast_to(ln_b.astype(f32), (8, U))

    out = _finalize(agg, Wt0.astype(f32), Wt1.astype(f32), Wt2.astype(f32),
                    Ws1.astype(f32), bs12, Ws2p, bs2p2, lng2, lnb2)
    return out[:N].reshape(N, 3, 3, U)


# speedup vs baseline: 1.0506x; 1.0506x over previous
import functools

import jax
import jax.numpy as jnp
from jax import lax
from jax.experimental import pallas as pl
from jax.experimental.pallas import tpu as pltpu
from jax.experimental.pallas import tpu_sc as plsc

U = 128
E = 160000
N = 10000
NPAD = 10240
NB = 128
NNB = NPAD // NB
B = 640
NEB = E // B
W = NEB + NNB - 1
NB2 = 200
NPB = 1280
CUTOFF = 5.0

SC_NC = 2
SC_NS = 16
SC_NW = SC_NC * SC_NS
EPW = E // SC_NW
SC_CH = 128
SC_NFULL = EPW // SC_CH
SC_TAIL = EPW - SC_NFULL * SC_CH


def _node_tables(z8, emb_pad, Wemb2):
    def body(z_ref, emb_ref, w_ref, xl_ref, xr_ref):
        ew_l = lax.dot_general(emb_ref[...], w_ref[:, :U], (((1,), (1,)), ((), ())),
                               preferred_element_type=jnp.float32)
        ew_r = lax.dot_general(emb_ref[...], w_ref[:, U:], (((1,), (1,)), ((), ())),
                               preferred_element_type=jnp.float32)
        zrow = z_ref[0:1, :]
        tids = lax.broadcasted_iota(jnp.int32, (U, NPB), 0).astype(jnp.float32)
        P = (tids == zrow).astype(jnp.float32)
        xl_ref[...] = lax.dot_general(P, ew_l, (((0,), (0,)), ((), ())),
                                      preferred_element_type=jnp.float32)
        xr_ref[...] = lax.dot_general(P, ew_r, (((0,), (0,)), ((), ())),
                                      preferred_element_type=jnp.float32)
    return pl.pallas_call(
        body,
        grid=(NPAD // NPB,),
        in_specs=[
            pl.BlockSpec((8, NPB), lambda i: (0, i)),
            pl.BlockSpec((U, U), lambda i: (0, 0)),
            pl.BlockSpec((U, 2 * U), lambda i: (0, 0)),
        ],
        out_specs=[pl.BlockSpec((NPB, U), lambda i: (i, 0)),
                   pl.BlockSpec((NPB, U), lambda i: (i, 0))],
        out_shape=[jax.ShapeDtypeStruct((NPAD, U), jnp.float32),
                   jax.ShapeDtypeStruct((NPAD, U), jnp.float32)],
    )(z8, emb_pad, Wemb2)


def _sc_gather(ei, xl, xr):
    mesh = plsc.VectorSubcoreMesh(core_axis_name="c", subcore_axis_name="s")

    @functools.partial(
        pl.kernel,
        out_type=jax.ShapeDtypeStruct((E, U), jnp.float32),
        mesh=mesh,
        scratch_types=[
            pltpu.VMEM((EPW,), jnp.int32),
            pltpu.VMEM((EPW,), jnp.int32),
            pltpu.VMEM((SC_CH, U), jnp.float32),
            pltpu.VMEM((SC_CH, U), jnp.float32),
            pltpu.VMEM((SC_CH, U), jnp.float32),
            pltpu.VMEM((SC_CH, U), jnp.float32),
            pltpu.VMEM((SC_TAIL, U), jnp.float32),
            pltpu.VMEM((SC_TAIL, U), jnp.float32),
            pltpu.SemaphoreType.DMA,
            pltpu.SemaphoreType.DMA,
            pltpu.SemaphoreType.DMA,
            pltpu.SemaphoreType.DMA,
        ],
    )
    def k(ei_hbm, xl_hbm, xr_hbm, out_hbm,
          is_v, id_v, al, ar, bl, br, tl, tr, sAl, sAr, sBl, sBr):
        wid = lax.axis_index("s") * SC_NC + lax.axis_index("c")
        base = wid * EPW
        pltpu.sync_copy(ei_hbm.at[pl.ds(base, EPW)], is_v)
        pltpu.sync_copy(ei_hbm.at[pl.ds(E + base, EPW)], id_v)

        def fire(c, bufl, bufr, sl, sr):
            off = c * SC_CH
            pltpu.async_copy(xl_hbm.at[is_v.at[pl.ds(off, SC_CH)]], bufl, sl)
            pltpu.async_copy(xr_hbm.at[id_v.at[pl.ds(off, SC_CH)]], bufr, sr)

        def addrows(bufl, bufr, n):
            def row(i, _):
                for c in range(U // 16):
                    s = pl.ds(c * 16, 16)
                    bufl[i, s] = bufl[i, s] + bufr[i, s]
                return 0
            lax.fori_loop(0, n, row, 0)

        def drain_store(c, bufl, bufr, sl, sr):
            off = c * SC_CH
            pltpu.make_async_copy(xl_hbm.at[is_v.at[pl.ds(off, SC_CH)]], bufl, sl).wait()
            pltpu.make_async_copy(xr_hbm.at[id_v.at[pl.ds(off, SC_CH)]], bufr, sr).wait()
            addrows(bufl, bufr, SC_CH)
            pltpu.sync_copy(bufl, out_hbm.at[pl.ds(base + off, SC_CH)])

        fire(0, al, ar, sAl, sAr)

        def body(tt, _):
            e = 2 * tt
            fire(e + 1, bl, br, sBl, sBr)
            drain_store(e, al, ar, sAl, sAr)
            fire(e + 2, al, ar, sAl, sAr)
            drain_store(e + 1, bl, br, sBl, sBr)
            return 0

        lax.fori_loop(0, (SC_NFULL - 1) // 2, body, 0)
        drain_store(SC_NFULL - 1, al, ar, sAl, sAr)
        off_t = SC_NFULL * SC_CH
        pltpu.async_copy(xl_hbm.at[is_v.at[pl.ds(off_t, SC_TAIL)]], tl, sAl)
        pltpu.async_copy(xr_hbm.at[id_v.at[pl.ds(off_t, SC_TAIL)]], tr, sAr)
        pltpu.make_async_copy(xl_hbm.at[is_v.at[pl.ds(off_t, SC_TAIL)]], tl, sAl).wait()
        pltpu.make_async_copy(xr_hbm.at[id_v.at[pl.ds(off_t, SC_TAIL)]], tr, sAr).wait()
        addrows(tl, tr, SC_TAIL)
        pltpu.sync_copy(tl, out_hbm.at[pl.ds(base + off_t, SC_TAIL)])

    return k(ei, xl, xr)


def _agg_body(nbm_ref, ebm_ref, eattr_ref, sp_ref, zp_ref,
              wdist_ref, bdist_ref, bemb2_ref, out_ref):
    w = pl.program_id(0)
    nb = nbm_ref[w]
    prev = nbm_ref[jnp.maximum(w - 1, 0)]

    @pl.when(jnp.logical_or(w == 0, nb != prev))
    def _():
        out_ref[...] = jnp.zeros_like(out_ref)

    sp = sp_ref[...]
    vx0, vy0, vz0 = sp[0:1, :], sp[1:2, :], sp[2:3, :]
    wgt, dstf = sp[3:4, :], sp[4:5, :]
    inv = 1.0 / jnp.maximum(jnp.sqrt(vx0 * vx0 + vy0 * vy0 + vz0 * vz0), 1e-6)
    vx, vy, vz = vx0 * inv, vy0 * inv, vz0 * inv
    C = 0.5 * (jnp.cos(jnp.pi / CUTOFF * wgt) + 1.0) * (wgt < CUTOFF)
    sxx, syy, szz = vx * vx, vy * vy, vz * vz
    tr3 = (sxx + syy + szz) * (1.0 / 3.0)
    cs = (C, C * vx, C * vy, C * vz,
          C * (sxx - tr3), C * (vx * vy), C * (vx * vz),
          C * (syy - tr3), C * (vy * vz), C * (szz - tr3))

    ea = lax.dot_general(eattr_ref[...], wdist_ref[...], (((1,), (1,)), ((), ())),
                         preferred_element_type=jnp.float32) + bdist_ref[0:1, :]
    Zij = zp_ref[...] + bemb2_ref[0:1, :]
    bf16 = jnp.bfloat16
    f0 = (ea[:, :U] * Zij).astype(bf16)
    f1 = (ea[:, U:2 * U] * Zij).astype(bf16)
    f2 = (ea[:, 2 * U:] * Zij).astype(bf16)

    ids = (nb * NB + lax.broadcasted_iota(jnp.int32, (NB, B), 0)).astype(jnp.float32)
    hit = ids == dstf
    P0 = jnp.where(hit, cs[0], 0.0).astype(bf16)
    P1 = jnp.concatenate([jnp.where(hit, cs[k], 0.0) for k in (1, 2, 3)],
                         axis=0).astype(bf16)
    P2 = jnp.concatenate([jnp.where(hit, cs[k], 0.0) for k in range(4, 10)],
                         axis=0).astype(bf16)
    dn = (((1,), (0,)), ((), ()))
    out_ref[0, :NB, :] += lax.dot_general(P0, f0, dn,
                                          preferred_element_type=jnp.float32)
    out_ref[0, NB:4 * NB, :] += lax.dot_general(P1, f1, dn,
                                                preferred_element_type=jnp.float32)
    out_ref[0, 4 * NB:, :] += lax.dot_general(P2, f2, dn,
                                              preferred_element_type=jnp.float32)


def _aggregate(nb_map, eb_map, eattr, sp, zp, Wdist, bdist2, bemb22):
    grid_spec = pltpu.PrefetchScalarGridSpec(
        num_scalar_prefetch=2,
        grid=(W,),
        in_specs=[
            pl.BlockSpec((B, 16), lambda w, nbm, ebm: (ebm[w], 0)),
            pl.BlockSpec((8, B), lambda w, nbm, ebm: (0, ebm[w])),
            pl.BlockSpec((B, U), lambda w, nbm, ebm: (ebm[w], 0)),
            pl.BlockSpec((3 * U, 16), lambda w, nbm, ebm: (0, 0)),
            pl.BlockSpec((8, 3 * U), lambda w, nbm, ebm: (0, 0)),
            pl.BlockSpec((8, U), lambda w, nbm, ebm: (0, 0)),
        ],
        out_specs=pl.BlockSpec((1, 10 * NB, U), lambda w, nbm, ebm: (nbm[w], 0, 0)),
    )
    return pl.pallas_call(
        _agg_body,
        grid_spec=grid_spec,
        out_shape=jax.ShapeDtypeStruct((NNB, 10 * NB, U), jnp.float32),
        compiler_params=pltpu.CompilerParams(dimension_semantics=("arbitrary",)),
    )(nb_map, eb_map, eattr, sp, zp, Wdist, bdist2, bemb22)


def _final_body(agg_ref, wt0_ref, wt1_ref, wt2_ref, ws1_ref, bs1_ref,
                ws2_ref, bs2_ref, lng_ref, lnb_ref, out_ref):
    a = agg_ref[0]
    Isum = a[:NB]
    ax, ay, az = a[NB:2 * NB], a[2 * NB:3 * NB], a[3 * NB:4 * NB]
    sxx, sxy, sxz = a[4 * NB:5 * NB], a[5 * NB:6 * NB], a[6 * NB:7 * NB]
    syy, syz, szz = a[7 * NB:8 * NB], a[8 * NB:9 * NB], a[9 * NB:]

    x00, x11, x22 = Isum + sxx, Isum + syy, Isum + szz
    nrm = (x00 * x00 + x11 * x11 + x22 * x22
           + (sxy - az) ** 2 + (sxy + az) ** 2
           + (sxz + ay) ** 2 + (sxz - ay) ** 2
           + (syz - ax) ** 2 + (syz + ax) ** 2)
    mu = jnp.mean(nrm, axis=1, keepdims=True)
    var = jnp.mean((nrm - mu) ** 2, axis=1, keepdims=True)
    h = (nrm - mu) / jnp.sqrt(var + 1e-5) * lng_ref[0:1, :] + lnb_ref[0:1, :]

    h1 = lax.dot_general(h, ws1_ref[...], (((1,), (1,)), ((), ())),
                         preferred_element_type=jnp.float32) + bs1_ref[0:1, :]
    h1 = h1 * (1.0 / (1.0 + jnp.exp(-h1)))
    h2 = lax.dot_general(h1, ws2_ref[...], (((1,), (1,)), ((), ())),
                         preferred_element_type=jnp.float32) + bs2_ref[0:1, :]
    h2 = h2 * (1.0 / (1.0 + jnp.exp(-h2)))
    nI, nA, nS = h2[:, :U], h2[:, U:2 * U], h2[:, 2 * U:]

    tI = lax.dot_general(Isum, wt0_ref[...], (((1,), (1,)), ((), ())),
                         preferred_element_type=jnp.float32)
    tA = lax.dot_general(a[NB:4 * NB], wt1_ref[...], (((1,), (1,)), ((), ())),
                         preferred_element_type=jnp.float32)
    tax, tay, taz = tA[:NB], tA[NB:2 * NB], tA[2 * NB:]
    tS = lax.dot_general(a[4 * NB:], wt2_ref[...], (((1,), (1,)), ((), ())),
                         preferred_element_type=jnp.float32)
    tsxx, tsxy, tsxz = tS[:NB], tS[NB:2 * NB], tS[2 * NB:3 * NB]
    tsyy, tsyz, tszz = tS[3 * NB:4 * NB], tS[4 * NB:5 * NB], tS[5 * NB:]

    dI = tI * nI
    out_ref[...] = jnp.concatenate([
        dI + tsxx * nS,
        -taz * nA + tsxy * nS,
        tay * nA + tsxz * nS,
        taz * nA + tsxy * nS,
        dI + tsyy * nS,
        -tax * nA + tsyz * nS,
        -tay * nA + tsxz * nS,
        tax * nA + tsyz * nS,
        dI + tszz * nS,
    ], axis=1)


def _finalize(agg, Wt0, Wt1, Wt2, Ws1, bs12, Ws2p, bs2p2, lng2, lnb2):
    return pl.pallas_call(
        _final_body,
        grid=((N + NB - 1) // NB,),
        in_specs=[
            pl.BlockSpec((1, 10 * NB, U), lambda i: (i, 0, 0)),
            pl.BlockSpec((U, U), lambda i: (0, 0)),
            pl.BlockSpec((U, U), lambda i: (0, 0)),
            pl.BlockSpec((U, U), lambda i: (0, 0)),
            pl.BlockSpec((2 * U, U), lambda i: (0, 0)),
            pl.BlockSpec((8, 2 * U), lambda i: (0, 0)),
            pl.BlockSpec((3 * U, 2 * U), lambda i: (0, 0)),
            pl.BlockSpec((8, 3 * U), lambda i: (0, 0)),
            pl.BlockSpec((8, U), lambda i: (0, 0)),
            pl.BlockSpec((8, U), lambda i: (0, 0)),
        ],
        out_specs=pl.BlockSpec((NB, 9 * U), lambda i: (i, 0)),
        out_shape=jax.ShapeDtypeStruct((N, 9 * U), jnp.float32),
    )(agg, Wt0, Wt1, Wt2, Ws1, bs12, Ws2p, bs2p2, lng2, lnb2)


def kernel(z, edge_index, edge_weight, edge_vec, edge_attr, col_data,
           col_indptr, emb, Wdist, bdist, Wemb2, bemb2, Wt0, Wt1, Wt2,
           Ws1, bs1, Ws2, bs2, ln_g, ln_b):
    f32 = jnp.float32
    src = edge_index[0].astype(jnp.int32)
    dst = edge_index[1].astype(jnp.int32)

    emb_pad = jnp.pad(emb.astype(f32), ((0, U - emb.shape[0]), (0, 0)))
    zp = jnp.pad(z.astype(f32), (0, NPAD - N))
    z8 = jnp.broadcast_to(zp[None, :], (8, NPAD))
    xl, xr = _node_tables(z8, emb_pad, Wemb2.astype(f32))
    zp = _sc_gather(edge_index.astype(jnp.int32).reshape(2 * E), xl, xr)

    sp = jnp.concatenate([
        edge_vec.astype(f32).T,
        edge_weight.astype(f32)[None, :],
        dst.astype(f32)[None, :],
        jnp.zeros((3, E), f32),
    ], axis=0)

    lnb = dst[B - 1::B] // NB
    b = lnb.at[-1].set(NNB - 1)
    bprev = jnp.concatenate([jnp.zeros((1,), jnp.int32), b[:-1]])
    cnt = b - bprev + 1
    eb_map = jnp.repeat(jnp.arange(NEB, dtype=jnp.int32), cnt,
                        total_repeat_length=W)
    start = jnp.cumsum(cnt) - cnt
    nb_map = (bprev[eb_map] + jnp.arange(W, dtype=jnp.int32)
              - start[eb_map]).astype(jnp.int32)

    bdist2 = jnp.broadcast_to(bdist.astype(f32), (8, 3 * U))
    bemb22 = jnp.broadcast_to(bemb2.astype(f32), (8, U))
    agg = _aggregate(nb_map, eb_map, edge_attr.astype(f32), sp, zp,
                     Wdist.astype(f32), bdist2, bemb22)

    perm = jnp.concatenate([jnp.arange(U) * 3, jnp.arange(U) * 3 + 1,
                            jnp.arange(U) * 3 + 2])
    Ws2p = Ws2[perm].astype(f32)
    bs2p2 = jnp.broadcast_to(bs2[perm].astype(f32), (8, 3 * U))
    bs12 = jnp.broadcast_to(bs1.astype(f32), (8, 2 * U))
    lng2 = jnp.broadcast_to(ln_g.astype(f32), (8, U))
    lnb2 = jnp.broadcast_to(ln_b.astype(f32), (8, U))

    out = _finalize(agg, Wt0.astype(f32), Wt1.astype(f32), Wt2.astype(f32),
                    Ws1.astype(f32), bs12, Ws2p, bs2p2, lng2, lnb2)
    return out.reshape(N, 3, 3, U)


# speedup vs baseline: 1.0827x; 1.0305x over previous
import functools

import jax
import jax.numpy as jnp
from jax import lax
from jax.experimental import pallas as pl
from jax.experimental.pallas import tpu as pltpu
from jax.experimental.pallas import tpu_sc as plsc

U = 128
E = 160000
N = 10000
NPAD = 10240
NB = 128
NNB = NPAD // NB
B = 1280
NEB = E // B
W = NEB + NNB - 1
NB2 = 200
NPB = 1280
CUTOFF = 5.0

SC_NC = 2
SC_NS = 16
SC_NW = SC_NC * SC_NS
EPW = E // SC_NW
SC_CH = 128
SC_NFULL = EPW // SC_CH
SC_TAIL = EPW - SC_NFULL * SC_CH


def _node_tables(z8, emb_pad, Wemb2):
    def body(z_ref, emb_ref, w_ref, xl_ref, xr_ref):
        ew_l = lax.dot_general(emb_ref[...], w_ref[:, :U], (((1,), (1,)), ((), ())),
                               preferred_element_type=jnp.float32)
        ew_r = lax.dot_general(emb_ref[...], w_ref[:, U:], (((1,), (1,)), ((), ())),
                               preferred_element_type=jnp.float32)
        zrow = z_ref[0:1, :]
        tids = lax.broadcasted_iota(jnp.int32, (U, NPB), 0).astype(jnp.float32)
        P = (tids == zrow).astype(jnp.float32)
        xl_ref[...] = lax.dot_general(P, ew_l, (((0,), (0,)), ((), ())),
                                      preferred_element_type=jnp.float32)
        xr_ref[...] = lax.dot_general(P, ew_r, (((0,), (0,)), ((), ())),
                                      preferred_element_type=jnp.float32)
    return pl.pallas_call(
        body,
        grid=(NPAD // NPB,),
        in_specs=[
            pl.BlockSpec((8, NPB), lambda i: (0, i)),
            pl.BlockSpec((U, U), lambda i: (0, 0)),
            pl.BlockSpec((U, 2 * U), lambda i: (0, 0)),
        ],
        out_specs=[pl.BlockSpec((NPB, U), lambda i: (i, 0)),
                   pl.BlockSpec((NPB, U), lambda i: (i, 0))],
        out_shape=[jax.ShapeDtypeStruct((NPAD, U), jnp.float32),
                   jax.ShapeDtypeStruct((NPAD, U), jnp.float32)],
    )(z8, emb_pad, Wemb2)


def _sc_gather(ei, xl, xr):
    mesh = plsc.VectorSubcoreMesh(core_axis_name="c", subcore_axis_name="s")

    @functools.partial(
        pl.kernel,
        out_type=jax.ShapeDtypeStruct((E, U), jnp.float32),
        mesh=mesh,
        scratch_types=[
            pltpu.VMEM((EPW,), jnp.int32),
            pltpu.VMEM((EPW,), jnp.int32),
            pltpu.VMEM((SC_CH, U), jnp.float32),
            pltpu.VMEM((SC_CH, U), jnp.float32),
            pltpu.VMEM((SC_CH, U), jnp.float32),
            pltpu.VMEM((SC_CH, U), jnp.float32),
            pltpu.VMEM((SC_TAIL, U), jnp.float32),
            pltpu.VMEM((SC_TAIL, U), jnp.float32),
            pltpu.SemaphoreType.DMA,
            pltpu.SemaphoreType.DMA,
            pltpu.SemaphoreType.DMA,
            pltpu.SemaphoreType.DMA,
        ],
    )
    def k(ei_hbm, xl_hbm, xr_hbm, out_hbm,
          is_v, id_v, al, ar, bl, br, tl, tr, sAl, sAr, sBl, sBr):
        wid = lax.axis_index("s") * SC_NC + lax.axis_index("c")
        base = wid * EPW
        pltpu.sync_copy(ei_hbm.at[pl.ds(base, EPW)], is_v)
        pltpu.sync_copy(ei_hbm.at[pl.ds(E + base, EPW)], id_v)

        def fire(c, bufl, bufr, sl, sr):
            off = c * SC_CH
            pltpu.async_copy(xl_hbm.at[is_v.at[pl.ds(off, SC_CH)]], bufl, sl)
            pltpu.async_copy(xr_hbm.at[id_v.at[pl.ds(off, SC_CH)]], bufr, sr)

        def addrows(bufl, bufr, n):
            def row(i, _):
                for c in range(U // 16):
                    s = pl.ds(c * 16, 16)
                    bufl[i, s] = bufl[i, s] + bufr[i, s]
                return 0
            lax.fori_loop(0, n, row, 0)

        def drain_store(c, bufl, bufr, sl, sr):
            off = c * SC_CH
            pltpu.make_async_copy(xl_hbm.at[is_v.at[pl.ds(off, SC_CH)]], bufl, sl).wait()
            pltpu.make_async_copy(xr_hbm.at[id_v.at[pl.ds(off, SC_CH)]], bufr, sr).wait()
            addrows(bufl, bufr, SC_CH)
            pltpu.sync_copy(bufl, out_hbm.at[pl.ds(base + off, SC_CH)])

        fire(0, al, ar, sAl, sAr)

        def body(tt, _):
            e = 2 * tt
            fire(e + 1, bl, br, sBl, sBr)
            drain_store(e, al, ar, sAl, sAr)
            fire(e + 2, al, ar, sAl, sAr)
            drain_store(e + 1, bl, br, sBl, sBr)
            return 0

        lax.fori_loop(0, (SC_NFULL - 1) // 2, body, 0)
        drain_store(SC_NFULL - 1, al, ar, sAl, sAr)
        off_t = SC_NFULL * SC_CH
        pltpu.async_copy(xl_hbm.at[is_v.at[pl.ds(off_t, SC_TAIL)]], tl, sAl)
        pltpu.async_copy(xr_hbm.at[id_v.at[pl.ds(off_t, SC_TAIL)]], tr, sAr)
        pltpu.make_async_copy(xl_hbm.at[is_v.at[pl.ds(off_t, SC_TAIL)]], tl, sAl).wait()
        pltpu.make_async_copy(xr_hbm.at[id_v.at[pl.ds(off_t, SC_TAIL)]], tr, sAr).wait()
        addrows(tl, tr, SC_TAIL)
        pltpu.sync_copy(tl, out_hbm.at[pl.ds(base + off_t, SC_TAIL)])

    return k(ei, xl, xr)


def _agg_body(nbm_ref, ebm_ref, eattr_ref, sp_ref, zp_ref,
              wdist_ref, bdist_ref, bemb2_ref, out_ref):
    w = pl.program_id(0)
    nb = nbm_ref[w]
    prev = nbm_ref[jnp.maximum(w - 1, 0)]

    @pl.when(jnp.logical_or(w == 0, nb != prev))
    def _():
        out_ref[...] = jnp.zeros_like(out_ref)

    sp = sp_ref[...]
    vx0, vy0, vz0 = sp[0:1, :], sp[1:2, :], sp[2:3, :]
    wgt, dstf = sp[3:4, :], sp[4:5, :]
    inv = 1.0 / jnp.maximum(jnp.sqrt(vx0 * vx0 + vy0 * vy0 + vz0 * vz0), 1e-6)
    vx, vy, vz = vx0 * inv, vy0 * inv, vz0 * inv
    C = 0.5 * (jnp.cos(jnp.pi / CUTOFF * wgt) + 1.0) * (wgt < CUTOFF)
    sxx, syy, szz = vx * vx, vy * vy, vz * vz
    tr3 = (sxx + syy + szz) * (1.0 / 3.0)
    cs = (C, C * vx, C * vy, C * vz,
          C * (sxx - tr3), C * (vx * vy), C * (vx * vz),
          C * (syy - tr3), C * (vy * vz), C * (szz - tr3))

    ea = lax.dot_general(eattr_ref[...], wdist_ref[...], (((1,), (1,)), ((), ())),
                         preferred_element_type=jnp.float32) + bdist_ref[0:1, :]
    Zij = zp_ref[...] + bemb2_ref[0:1, :]
    bf16 = jnp.bfloat16
    f0 = (ea[:, :U] * Zij).astype(bf16)
    f1 = (ea[:, U:2 * U] * Zij).astype(bf16)
    f2 = (ea[:, 2 * U:] * Zij).astype(bf16)

    ids = (nb * NB + lax.broadcasted_iota(jnp.int32, (NB, B), 0)).astype(jnp.float32)
    hit = ids == dstf
    P0 = jnp.where(hit, cs[0], 0.0).astype(bf16)
    P1 = jnp.concatenate([jnp.where(hit, cs[k], 0.0) for k in (1, 2, 3)],
                         axis=0).astype(bf16)
    P2 = jnp.concatenate([jnp.where(hit, cs[k], 0.0) for k in range(4, 10)],
                         axis=0).astype(bf16)
    dn = (((1,), (0,)), ((), ()))
    out_ref[0, :NB, :] += lax.dot_general(P0, f0, dn,
                                          preferred_element_type=jnp.float32)
    out_ref[0, NB:4 * NB, :] += lax.dot_general(P1, f1, dn,
                                                preferred_element_type=jnp.float32)
    out_ref[0, 4 * NB:, :] += lax.dot_general(P2, f2, dn,
                                              preferred_element_type=jnp.float32)


def _aggregate(nb_map, eb_map, eattr, sp, zp, Wdist, bdist2, bemb22):
    grid_spec = pltpu.PrefetchScalarGridSpec(
        num_scalar_prefetch=2,
        grid=(W,),
        in_specs=[
            pl.BlockSpec((B, 16), lambda w, nbm, ebm: (ebm[w], 0)),
            pl.BlockSpec((8, B), lambda w, nbm, ebm: (0, ebm[w])),
            pl.BlockSpec((B, U), lambda w, nbm, ebm: (ebm[w], 0)),
            pl.BlockSpec((3 * U, 16), lambda w, nbm, ebm: (0, 0)),
            pl.BlockSpec((8, 3 * U), lambda w, nbm, ebm: (0, 0)),
            pl.BlockSpec((8, U), lambda w, nbm, ebm: (0, 0)),
        ],
        out_specs=pl.BlockSpec((1, 10 * NB, U), lambda w, nbm, ebm: (nbm[w], 0, 0)),
    )
    return pl.pallas_call(
        _agg_body,
        grid_spec=grid_spec,
        out_shape=jax.ShapeDtypeStruct((NNB, 10 * NB, U), jnp.float32),
        compiler_params=pltpu.CompilerParams(dimension_semantics=("arbitrary",)),
    )(nb_map, eb_map, eattr, sp, zp, Wdist, bdist2, bemb22)


def _final_body(agg_ref, wt0_ref, wt1_ref, wt2_ref, ws1_ref, bs1_ref,
                ws2_ref, bs2_ref, lng_ref, lnb_ref, out_ref):
    a = agg_ref[0]
    Isum = a[:NB]
    ax, ay, az = a[NB:2 * NB], a[2 * NB:3 * NB], a[3 * NB:4 * NB]
    sxx, sxy, sxz = a[4 * NB:5 * NB], a[5 * NB:6 * NB], a[6 * NB:7 * NB]
    syy, syz, szz = a[7 * NB:8 * NB], a[8 * NB:9 * NB], a[9 * NB:]

    x00, x11, x22 = Isum + sxx, Isum + syy, Isum + szz
    nrm = (x00 * x00 + x11 * x11 + x22 * x22
           + (sxy - az) ** 2 + (sxy + az) ** 2
           + (sxz + ay) ** 2 + (sxz - ay) ** 2
           + (syz - ax) ** 2 + (syz + ax) ** 2)
    mu = jnp.mean(nrm, axis=1, keepdims=True)
    var = jnp.mean((nrm - mu) ** 2, axis=1, keepdims=True)
    h = (nrm - mu) / jnp.sqrt(var + 1e-5) * lng_ref[0:1, :] + lnb_ref[0:1, :]

    h1 = lax.dot_general(h, ws1_ref[...], (((1,), (1,)), ((), ())),
                         preferred_element_type=jnp.float32) + bs1_ref[0:1, :]
    h1 = h1 * (1.0 / (1.0 + jnp.exp(-h1)))
    h2 = lax.dot_general(h1, ws2_ref[...], (((1,), (1,)), ((), ())),
                         preferred_element_type=jnp.float32) + bs2_ref[0:1, :]
    h2 = h2 * (1.0 / (1.0 + jnp.exp(-h2)))
    nI, nA, nS = h2[:, :U], h2[:, U:2 * U], h2[:, 2 * U:]

    tI = lax.dot_general(Isum, wt0_ref[...], (((1,), (1,)), ((), ())),
                         preferred_element_type=jnp.float32)
    tA = lax.dot_general(a[NB:4 * NB], wt1_ref[...], (((1,), (1,)), ((), ())),
                         preferred_element_type=jnp.float32)
    tax, tay, taz = tA[:NB], tA[NB:2 * NB], tA[2 * NB:]
    tS = lax.dot_general(a[4 * NB:], wt2_ref[...], (((1,), (1,)), ((), ())),
                         preferred_element_type=jnp.float32)
    tsxx, tsxy, tsxz = tS[:NB], tS[NB:2 * NB], tS[2 * NB:3 * NB]
    tsyy, tsyz, tszz = tS[3 * NB:4 * NB], tS[4 * NB:5 * NB], tS[5 * NB:]

    dI = tI * nI
    out_ref[...] = jnp.concatenate([
        dI + tsxx * nS,
        -taz * nA + tsxy * nS,
        tay * nA + tsxz * nS,
        taz * nA + tsxy * nS,
        dI + tsyy * nS,
        -tax * nA + tsyz * nS,
        -tay * nA + tsxz * nS,
        tax * nA + tsyz * nS,
        dI + tszz * nS,
    ], axis=1)


def _finalize(agg, Wt0, Wt1, Wt2, Ws1, bs12, Ws2p, bs2p2, lng2, lnb2):
    return pl.pallas_call(
        _final_body,
        grid=((N + NB - 1) // NB,),
        in_specs=[
            pl.BlockSpec((1, 10 * NB, U), lambda i: (i, 0, 0)),
            pl.BlockSpec((U, U), lambda i: (0, 0)),
            pl.BlockSpec((U, U), lambda i: (0, 0)),
            pl.BlockSpec((U, U), lambda i: (0, 0)),
            pl.BlockSpec((2 * U, U), lambda i: (0, 0)),
            pl.BlockSpec((8, 2 * U), lambda i: (0, 0)),
            pl.BlockSpec((3 * U, 2 * U), lambda i: (0, 0)),
            pl.BlockSpec((8, 3 * U), lambda i: (0, 0)),
            pl.BlockSpec((8, U), lambda i: (0, 0)),
            pl.BlockSpec((8, U), lambda i: (0, 0)),
        ],
        out_specs=pl.BlockSpec((NB, 9 * U), lambda i: (i, 0)),
        out_shape=jax.ShapeDtypeStruct((N, 9 * U), jnp.float32),
    )(agg, Wt0, Wt1, Wt2, Ws1, bs12, Ws2p, bs2p2, lng2, lnb2)


def kernel(z, edge_index, edge_weight, edge_vec, edge_attr, col_data,
           col_indptr, emb, Wdist, bdist, Wemb2, bemb2, Wt0, Wt1, Wt2,
           Ws1, bs1, Ws2, bs2, ln_g, ln_b):
    f32 = jnp.float32
    src = edge_index[0].astype(jnp.int32)
    dst = edge_index[1].astype(jnp.int32)

    emb_pad = jnp.pad(emb.astype(f32), ((0, U - emb.shape[0]), (0, 0)))
    zp = jnp.pad(z.astype(f32), (0, NPAD - N))
    z8 = jnp.broadcast_to(zp[None, :], (8, NPAD))
    xl, xr = _node_tables(z8, emb_pad, Wemb2.astype(f32))
    zp = _sc_gather(edge_index.astype(jnp.int32).reshape(2 * E), xl, xr)

    sp = jnp.concatenate([
        edge_vec.astype(f32).T,
        edge_weight.astype(f32)[None, :],
        dst.astype(f32)[None, :],
        jnp.zeros((3, E), f32),
    ], axis=0)

    lnb = dst[B - 1::B] // NB
    b = lnb.at[-1].set(NNB - 1)
    bprev = jnp.concatenate([jnp.zeros((1,), jnp.int32), b[:-1]])
    cnt = b - bprev + 1
    eb_map = jnp.repeat(jnp.arange(NEB, dtype=jnp.int32), cnt,
                        total_repeat_length=W)
    start = jnp.cumsum(cnt) - cnt
    nb_map = (bprev[eb_map] + jnp.arange(W, dtype=jnp.int32)
              - start[eb_map]).astype(jnp.int32)

    bdist2 = jnp.broadcast_to(bdist.astype(f32), (8, 3 * U))
    bemb22 = jnp.broadcast_to(bemb2.astype(f32), (8, U))
    agg = _aggregate(nb_map, eb_map, edge_attr.astype(f32), sp, zp,
                     Wdist.astype(f32), bdist2, bemb22)

    perm = jnp.concatenate([jnp.arange(U) * 3, jnp.arange(U) * 3 + 1,
                            jnp.arange(U) * 3 + 2])
    Ws2p = Ws2[perm].astype(f32)
    bs2p2 = jnp.broadcast_to(bs2[perm].astype(f32), (8, 3 * U))
    bs12 = jnp.broadcast_to(bs1.astype(f32), (8, 2 * U))
    lng2 = jnp.broadcast_to(ln_g.astype(f32), (8, U))
    lnb2 = jnp.broadcast_to(ln_b.astype(f32), (8, U))

    out = _finalize(agg, Wt0.astype(f32), Wt1.astype(f32), Wt2.astype(f32),
                    Ws1.astype(f32), bs12, Ws2p, bs2p2, lng2, lnb2)
    return out.reshape(N, 3, 3, U)
